# Initial kernel scaffold; baseline (speedup 1.0000x reference)
import functools

import jax
import jax.numpy as jnp
from jax import lax
from jax.experimental import pallas as pl
from jax.experimental.pallas import tpu as pltpu
from jax.experimental.pallas import tpu_sc as plsc

N = 10000
E = 320000
D = 128
NP = 10240
NC = 2
NS = 16
ER = E // 128
ERC = ER // NC
BASE = ERC // NS
EXTRA = ERC % NS
STG = 26
EPS = 1e-5

_sc_mesh = plsc.VectorSubcoreMesh(core_axis_name="c", subcore_axis_name="s")


@functools.partial(
    pl.kernel,
    out_type=jax.ShapeDtypeStruct((NC, NP), jnp.float32),
    mesh=_sc_mesh,
    scratch_types=[
        pltpu.VMEM((STG, 128), jnp.int32),
        pltpu.VMEM((STG, 128), jnp.float32),
        pltpu.VMEM((640,), jnp.float32),
        pltpu.VMEM_SHARED((NP,), jnp.float32),
    ],
)
def _deg_kernel(dst_hbm, w_hbm, out_hbm, dstv, wv, zbuf, shared):
    cid = lax.axis_index("c")
    sid = lax.axis_index("s")

    def _zero(i, _):
        zbuf[pl.ds(i * 16, 16)] = jnp.zeros((16,), jnp.float32)
        return 0

    lax.fori_loop(0, 40, _zero, 0)
    pltpu.sync_copy(zbuf, shared.at[pl.ds(sid * 640, 640)])
    plsc.subcore_barrier()

    r0 = cid * ERC + sid * BASE + jnp.minimum(sid, EXTRA)

    def _chunk(ci, _):
        base = r0 + ci * STG
        pltpu.sync_copy(dst_hbm.at[pl.ds(base, STG)], dstv)
        pltpu.sync_copy(w_hbm.at[pl.ds(base, STG)], wv)

        def _row(k, _):
            pltpu.sync_copy(wv.at[k], shared.at[dstv.at[k]], add=True)
            return 0

        lax.fori_loop(0, STG, _row, 0)
        return 0

    lax.fori_loop(0, BASE // STG, _chunk, 0)

    @pl.when(sid < EXTRA)
    def _tail():
        base = r0 + BASE
        pltpu.sync_copy(dst_hbm.at[pl.ds(base, 1)], dstv.at[pl.ds(0, 1)])
        pltpu.sync_copy(w_hbm.at[pl.ds(base, 1)], wv.at[pl.ds(0, 1)])
        pltpu.sync_copy(wv.at[0], shared.at[dstv.at[0]], add=True)

    plsc.subcore_barrier()
    pltpu.sync_copy(shared.at[pl.ds(sid * 640, 640)],
                    out_hbm.at[cid, pl.ds(sid * 640, 640)])


@functools.partial(
    pl.kernel,
    out_type=jax.ShapeDtypeStruct((NC, NP, D), jnp.float32),
    mesh=_sc_mesh,
    scratch_types=[
        pltpu.VMEM((STG, 128), jnp.int32),
        pltpu.VMEM((STG, 128), jnp.int32),
        pltpu.VMEM((STG, 128), jnp.float32),
        pltpu.VMEM((128, D), jnp.float32),
        pltpu.VMEM_SHARED((NP, D), jnp.float32),
        pltpu.SemaphoreType.DMA,
    ],
)
def _msg_kernel(y_hbm, src_hbm, dst_hbm, ew_hbm, out_hbm,
                srcv, dstv, ewv, rows, shared, sem):
    cid = lax.axis_index("c")
    sid = lax.axis_index("s")

    def _zrow(r, _):
        for c in range(8):
            rows[r, pl.ds(c * 16, 16)] = jnp.zeros((16,), jnp.float32)
        return 0

    lax.fori_loop(0, 128, _zrow, 0)
    for j in range(5):
        pltpu.sync_copy(rows, shared.at[pl.ds(sid * 640 + j * 128, 128)])
    plsc.subcore_barrier()

    r0 = cid * ERC + sid * BASE + jnp.minimum(sid, EXTRA)

    def _do_row(k):
        pltpu.async_copy(y_hbm.at[srcv.at[k]], rows, sem).wait()
        ksplat = jnp.broadcast_to(k, (16,)).astype(jnp.int32)

        def _group(gc, _):
            base16 = jnp.broadcast_to(gc * 16, (16,)).astype(jnp.int32)
            for j in range(16):
                e = gc * 16 + j
                bc = plsc.load_gather(ewv, [ksplat, base16 + j])
                for c in range(8):
                    rows[e, pl.ds(c * 16, 16)] = rows[e, pl.ds(c * 16, 16)] * bc
            return 0

        lax.fori_loop(0, 8, _group, 0)
        pltpu.sync_copy(rows, shared.at[dstv.at[k]], add=True)

    def _chunk(ci, _):
        base = r0 + ci * STG
        pltpu.sync_copy(src_hbm.at[pl.ds(base, STG)], srcv)
        pltpu.sync_copy(dst_hbm.at[pl.ds(base, STG)], dstv)
        pltpu.sync_copy(ew_hbm.at[pl.ds(base, STG)], ewv)

        def _row(k, _):
            _do_row(k)
            return 0

        lax.fori_loop(0, STG, _row, 0)
        return 0

    lax.fori_loop(0, BASE // STG, _chunk, 0)

    @pl.when(sid < EXTRA)
    def _tail():
        base = r0 + BASE
        pltpu.sync_copy(src_hbm.at[pl.ds(base, 1)], srcv.at[pl.ds(0, 1)])
        pltpu.sync_copy(dst_hbm.at[pl.ds(base, 1)], dstv.at[pl.ds(0, 1)])
        pltpu.sync_copy(ew_hbm.at[pl.ds(base, 1)], ewv.at[pl.ds(0, 1)])
        _do_row(0)

    plsc.subcore_barrier()
    pltpu.sync_copy(shared.at[pl.ds(sid * 640, 640)],
                    out_hbm.at[cid, pl.ds(sid * 640, 640)])


_BR = 1024


def _y_mm_body(x_ref, w_ref, degp_ref, y_ref):
    deg = degp_ref[0] + degp_ref[1] + 1.0
    dv = lax.rsqrt(deg)
    y_ref[...] = jnp.dot(x_ref[...], w_ref[...],
                         preferred_element_type=jnp.float32) * dv


def _y_mm(x, w, degs):
    grid = (NP // _BR,)
    return pl.pallas_call(
        _y_mm_body,
        grid=grid,
        in_specs=[
            pl.BlockSpec((_BR, D), lambda i: (i, 0)),
            pl.BlockSpec((D, D), lambda i: (0, 0)),
            pl.BlockSpec((NC, _BR, 1), lambda i: (0, i, 0)),
        ],
        out_specs=pl.BlockSpec((_BR, D), lambda i: (i, 0)),
        out_shape=jax.ShapeDtypeStruct((N, D), jnp.float32),
    )(x, w, degs)


def _fin1_body(p0, p1, y, degs, b, g, bt, w2, hbn_ref, y2_ref):
    dv = lax.rsqrt(degs[0] + degs[1] + 1.0)
    h = jnp.maximum(dv * (p0[...] + p1[...] + y[...]) + b[...], 0.0)
    m = jnp.mean(h, axis=0, keepdims=True)
    v = jnp.mean((h - m) ** 2, axis=0, keepdims=True)
    hbn = (h - m) * lax.rsqrt(v + EPS) * g[...] + bt[...]
    hbn_ref[...] = hbn
    y2_ref[...] = jnp.dot(hbn, w2[...], preferred_element_type=jnp.float32) * dv


def _fin1(p0, p1, y, degs, b, g, bt, w2):
    return pl.pallas_call(
        _fin1_body,
        out_shape=(jax.ShapeDtypeStruct((N, D), jnp.float32),
                   jax.ShapeDtypeStruct((N, D), jnp.float32)),
    )(p0, p1, y, degs, b, g, bt, w2)


def _fin2_body(x, h1, p0, p1, y2, degs, b2, g2, bt2, fw1, fb1, fw2, fb2,
               out_ref):
    dv = lax.rsqrt(degs[0] + degs[1] + 1.0)
    h = jnp.maximum(dv * (p0[...] + p1[...] + y2[...]) + b2[...], 0.0)
    m = jnp.mean(h, axis=0, keepdims=True)
    v = jnp.mean((h - m) ** 2, axis=0, keepdims=True)
    h2bn = (h - m) * lax.rsqrt(v + EPS) * g2[...] + bt2[...]
    w = fw1[...]
    t = (jnp.dot(x[...], w[0:D], preferred_element_type=jnp.float32)
         + jnp.dot(h1[...], w[D:2 * D], preferred_element_type=jnp.float32)
         + jnp.dot(h2bn, w[2 * D:3 * D], preferred_element_type=jnp.float32)
         + fb1[...])
    t = jnp.maximum(t, 0.0)
    out = jnp.dot(t, fw2[...], preferred_element_type=jnp.float32) + fb2[...]
    out_ref[...] = jnp.maximum(out, 0.0)


def _fin2(x, h1, p0, p1, y2, degs, b2, g2, bt2, fw1, fb1, fw2, fb2):
    return pl.pallas_call(
        _fin2_body,
        out_shape=jax.ShapeDtypeStruct((N, D), jnp.float32),
    )(x, h1, p0, p1, y2, degs, b2, g2, bt2, fw1, fb1, fw2, fb2)


def kernel(x, edge_index, weight, W1, b1, W2, b2, g1, bt1, g2, bt2,
           fW1, fb1, fW2, fb2):
    src2d = edge_index[0].reshape(ER, 128)
    dst2d = edge_index[1].reshape(ER, 128)
    ew2d = weight.reshape(ER, 128)

    degp = _deg_kernel(dst2d, ew2d)
    degs = degp.reshape(NC, NP, 1)
    degs_n = degs[:, :N]

    y1 = _y_mm(x, W1, degs)
    part1 = _msg_kernel(y1, src2d, dst2d, ew2d)
    h1, y2 = _fin1(part1[0, :N], part1[1, :N], y1, degs_n, b1, g1, bt1, W2)
    part2 = _msg_kernel(y2, src2d, dst2d, ew2d)
    out = _fin2(x, h1, part2[0, :N], part2[1, :N], y2, degs_n,
                b2, g2, bt2, fW1, fb1, fW2, fb2)
    return out


# speedup vs baseline: 16.1464x; 16.1464x over previous
import functools

import jax
import jax.numpy as jnp
from jax import lax
from jax.experimental import pallas as pl
from jax.experimental.pallas import tpu as pltpu
from jax.experimental.pallas import tpu_sc as plsc

N = 10000
E = 320000
D = 128
NP = 10240
NC = 2
NS = 16
EP = 327680
ERP = EP // 128
ERC = ERP // NC
TROWS = ERC // NS
STG = 40
EPS = 1e-5

_sc_mesh = plsc.VectorSubcoreMesh(core_axis_name="c", subcore_axis_name="s")


@functools.partial(
    pl.kernel,
    out_type=jax.ShapeDtypeStruct((NC, NP), jnp.float32),
    mesh=_sc_mesh,
    scratch_types=[
        pltpu.VMEM((STG, 128), jnp.int32),
        pltpu.VMEM((STG, 128), jnp.float32),
        pltpu.VMEM((640,), jnp.float32),
        pltpu.VMEM_SHARED((NP,), jnp.float32),
    ],
)
def _deg_kernel(dst_hbm, w_hbm, out_hbm, dstv, wv, zbuf, shared):
    cid = lax.axis_index("c")
    sid = lax.axis_index("s")

    def _zero(i, _):
        zbuf[pl.ds(i * 16, 16)] = jnp.zeros((16,), jnp.float32)
        return 0

    lax.fori_loop(0, 40, _zero, 0)
    pltpu.sync_copy(zbuf, shared.at[pl.ds(sid * 640, 640)])
    plsc.subcore_barrier()

    r0 = cid * ERC + sid * TROWS

    def _chunk(ci, _):
        base = pl.multiple_of(r0 + ci * STG, 8)
        pltpu.sync_copy(dst_hbm.at[pl.ds(base, STG)], dstv)
        pltpu.sync_copy(w_hbm.at[pl.ds(base, STG)], wv)

        def _row(k, _):
            pltpu.sync_copy(wv.at[k], shared.at[dstv.at[k]], add=True)
            return 0

        lax.fori_loop(0, STG, _row, 0)
        return 0

    lax.fori_loop(0, TROWS // STG, _chunk, 0)

    plsc.subcore_barrier()
    pltpu.sync_copy(shared.at[pl.ds(sid * 640, 640)],
                    out_hbm.at[cid, pl.ds(sid * 640, 640)])


@functools.partial(
    pl.kernel,
    out_type=jax.ShapeDtypeStruct((NC, NP, D), jnp.float32),
    mesh=_sc_mesh,
    scratch_types=[
        pltpu.VMEM((STG, 128), jnp.int32),
        pltpu.VMEM((STG, 128), jnp.int32),
        pltpu.VMEM((STG, 128), jnp.float32),
        pltpu.VMEM((128, D), jnp.float32),
        pltpu.VMEM_SHARED((NP, D), jnp.float32),
        pltpu.SemaphoreType.DMA,
    ],
)
def _msg_kernel(y_hbm, src_hbm, dst_hbm, ew_hbm, out_hbm,
                srcv, dstv, ewv, rows, shared, sem):
    cid = lax.axis_index("c")
    sid = lax.axis_index("s")

    def _zrow(r, _):
        for c in range(8):
            rows[r, pl.ds(c * 16, 16)] = jnp.zeros((16,), jnp.float32)
        return 0

    lax.fori_loop(0, 128, _zrow, 0)
    for j in range(5):
        pltpu.sync_copy(rows, shared.at[pl.ds(sid * 640 + j * 128, 128)])
    plsc.subcore_barrier()

    r0 = cid * ERC + sid * TROWS

    def _do_row(k):
        pltpu.async_copy(y_hbm.at[srcv.at[k]], rows, sem).wait()

        def _group(gc, _):
            ew16 = ewv[k, pl.ds(gc * 16, 16)]
            for j in range(16):
                e = gc * 16 + j
                bc = lax.gather(
                    ew16, jnp.full((16, 1), j, jnp.int32),
                    lax.GatherDimensionNumbers(offset_dims=(),
                                               collapsed_slice_dims=(0,),
                                               start_index_map=(0,)),
                    slice_sizes=(1,),
                    mode=lax.GatherScatterMode.PROMISE_IN_BOUNDS)
                for c in range(8):
                    rows[e, pl.ds(c * 16, 16)] = rows[e, pl.ds(c * 16, 16)] * bc
            return 0

        lax.fori_loop(0, 8, _group, 0)
        pltpu.sync_copy(rows, shared.at[dstv.at[k]], add=True)

    def _chunk(ci, _):
        base = pl.multiple_of(r0 + ci * STG, 8)
        pltpu.sync_copy(src_hbm.at[pl.ds(base, STG)], srcv)
        pltpu.sync_copy(dst_hbm.at[pl.ds(base, STG)], dstv)
        pltpu.sync_copy(ew_hbm.at[pl.ds(base, STG)], ewv)

        def _row(k, _):
            _do_row(k)
            return 0

        lax.fori_loop(0, STG, _row, 0)
        return 0

    lax.fori_loop(0, TROWS // STG, _chunk, 0)

    plsc.subcore_barrier()
    pltpu.sync_copy(shared.at[pl.ds(sid * 640, 640)],
                    out_hbm.at[cid, pl.ds(sid * 640, 640)])


_BR = 1024


def _y_mm_body(x_ref, w_ref, degp_ref, y_ref):
    deg = degp_ref[0] + degp_ref[1] + 1.0
    dv = lax.rsqrt(deg)
    y_ref[...] = jnp.dot(x_ref[...], w_ref[...],
                         preferred_element_type=jnp.float32) * dv


def _y_mm(x, w, degs):
    grid = (NP // _BR,)
    return pl.pallas_call(
        _y_mm_body,
        grid=grid,
        in_specs=[
            pl.BlockSpec((_BR, D), lambda i: (i, 0)),
            pl.BlockSpec((D, D), lambda i: (0, 0)),
            pl.BlockSpec((NC, _BR, 1), lambda i: (0, i, 0)),
        ],
        out_specs=pl.BlockSpec((_BR, D), lambda i: (i, 0)),
        out_shape=jax.ShapeDtypeStruct((N, D), jnp.float32),
    )(x, w, degs)


def _fin1_body(p0, p1, y, degs, b, g, bt, w2, hbn_ref, y2_ref):
    dv = lax.rsqrt(degs[0] + degs[1] + 1.0)
    h = jnp.maximum(dv * (p0[...] + p1[...] + y[...]) + b[...], 0.0)
    m = jnp.mean(h, axis=0, keepdims=True)
    v = jnp.mean((h - m) ** 2, axis=0, keepdims=True)
    hbn = (h - m) * lax.rsqrt(v + EPS) * g[...] + bt[...]
    hbn_ref[...] = hbn
    y2_ref[...] = jnp.dot(hbn, w2[...], preferred_element_type=jnp.float32) * dv


def _fin1(p0, p1, y, degs, b, g, bt, w2):
    return pl.pallas_call(
        _fin1_body,
        out_shape=(jax.ShapeDtypeStruct((N, D), jnp.float32),
                   jax.ShapeDtypeStruct((N, D), jnp.float32)),
    )(p0, p1, y, degs, b, g, bt, w2)


def _fin2_body(x, h1, p0, p1, y2, degs, b2, g2, bt2, fw1, fb1, fw2, fb2,
               out_ref):
    dv = lax.rsqrt(degs[0] + degs[1] + 1.0)
    h = jnp.maximum(dv * (p0[...] + p1[...] + y2[...]) + b2[...], 0.0)
    m = jnp.mean(h, axis=0, keepdims=True)
    v = jnp.mean((h - m) ** 2, axis=0, keepdims=True)
    h2bn = (h - m) * lax.rsqrt(v + EPS) * g2[...] + bt2[...]
    w = fw1[...]
    t = (jnp.dot(x[...], w[0:D], preferred_element_type=jnp.float32)
         + jnp.dot(h1[...], w[D:2 * D], preferred_element_type=jnp.float32)
         + jnp.dot(h2bn, w[2 * D:3 * D], preferred_element_type=jnp.float32)
         + fb1[...])
    t = jnp.maximum(t, 0.0)
    out = jnp.dot(t, fw2[...], preferred_element_type=jnp.float32) + fb2[...]
    out_ref[...] = jnp.maximum(out, 0.0)


def _fin2(x, h1, p0, p1, y2, degs, b2, g2, bt2, fw1, fb1, fw2, fb2):
    return pl.pallas_call(
        _fin2_body,
        out_shape=jax.ShapeDtypeStruct((N, D), jnp.float32),
    )(x, h1, p0, p1, y2, degs, b2, g2, bt2, fw1, fb1, fw2, fb2)


def kernel(x, edge_index, weight, W1, b1, W2, b2, g1, bt1, g2, bt2,
           fW1, fb1, fW2, fb2):
    npad = EP - E
    pad_idx = (jnp.arange(npad, dtype=jnp.int32) * 13) % N
    src2d = jnp.concatenate([edge_index[0], pad_idx]).reshape(ERP, 128)
    dst2d = jnp.concatenate([edge_index[1], pad_idx]).reshape(ERP, 128)
    ew2d = jnp.concatenate(
        [weight, jnp.zeros((npad,), jnp.float32)]).reshape(ERP, 128)

    degp = _deg_kernel(dst2d, ew2d)
    degs = degp.reshape(NC, NP, 1)
    degs_n = degs[:, :N]

    y1 = _y_mm(x, W1, degs)
    part1 = _msg_kernel(y1, src2d, dst2d, ew2d)
    h1, y2 = _fin1(part1[0, :N], part1[1, :N], y1, degs_n, b1, g1, bt1, W2)
    part2 = _msg_kernel(y2, src2d, dst2d, ew2d)
    out = _fin2(x, h1, part2[0, :N], part2[1, :N], y2, degs_n,
                b2, g2, bt2, fW1, fb1, fW2, fb2)
    return out


# speedup vs baseline: 23.8027x; 1.4742x over previous
import functools

import jax
import jax.numpy as jnp
from jax import lax
from jax.experimental import pallas as pl
from jax.experimental.pallas import tpu as pltpu
from jax.experimental.pallas import tpu_sc as plsc

N = 10000
E = 320000
D = 128
NP = 10240
NC = 2
NS = 16
EP = 327680
ERP = EP // 128
ERC = ERP // NC
TROWS = ERC // NS
STG = 40
EPS = 1e-5

_sc_mesh = plsc.VectorSubcoreMesh(core_axis_name="c", subcore_axis_name="s")


@functools.partial(
    pl.kernel,
    out_type=jax.ShapeDtypeStruct((NC, NP), jnp.float32),
    mesh=_sc_mesh,
    scratch_types=[
        pltpu.VMEM((STG, 128), jnp.int32),
        pltpu.VMEM((STG, 128), jnp.float32),
        pltpu.VMEM((640,), jnp.float32),
        pltpu.VMEM_SHARED((NP,), jnp.float32),
    ],
)
def _deg_kernel(dst_hbm, w_hbm, out_hbm, dstv, wv, zbuf, shared):
    cid = lax.axis_index("c")
    sid = lax.axis_index("s")

    def _zero(i, _):
        zbuf[pl.ds(i * 16, 16)] = jnp.zeros((16,), jnp.float32)
        return 0

    lax.fori_loop(0, 40, _zero, 0)
    pltpu.sync_copy(zbuf, shared.at[pl.ds(sid * 640, 640)])
    plsc.subcore_barrier()

    r0 = cid * ERC + sid * TROWS

    def _chunk(ci, _):
        base = pl.multiple_of(r0 + ci * STG, 8)
        pltpu.sync_copy(dst_hbm.at[pl.ds(base, STG)], dstv)
        pltpu.sync_copy(w_hbm.at[pl.ds(base, STG)], wv)

        def _row(k, _):
            pltpu.sync_copy(wv.at[k], shared.at[dstv.at[k]], add=True)
            return 0

        lax.fori_loop(0, STG, _row, 0)
        return 0

    lax.fori_loop(0, TROWS // STG, _chunk, 0)

    plsc.subcore_barrier()
    pltpu.sync_copy(shared.at[pl.ds(sid * 640, 640)],
                    out_hbm.at[cid, pl.ds(sid * 640, 640)])


@functools.partial(
    pl.kernel,
    out_type=jax.ShapeDtypeStruct((NC, NP, D), jnp.float32),
    mesh=_sc_mesh,
    scratch_types=[
        pltpu.VMEM((STG, 128), jnp.int32),
        pltpu.VMEM((STG, 128), jnp.int32),
        pltpu.VMEM((STG, 128), jnp.float32),
        pltpu.VMEM((128, D), jnp.float32),
        pltpu.VMEM((128, D), jnp.float32),
        pltpu.VMEM_SHARED((NP, D), jnp.float32),
        pltpu.SemaphoreType.DMA,
        pltpu.SemaphoreType.DMA,
    ],
)
def _msg_kernel(y_hbm, src_hbm, dst_hbm, ew_hbm, out_hbm,
                srcv, dstv, ewv, rows_a, rows_b, shared, sem_a, sem_b):
    cid = lax.axis_index("c")
    sid = lax.axis_index("s")

    def _zrow(r, _):
        for c in range(8):
            rows_a[r, pl.ds(c * 16, 16)] = jnp.zeros((16,), jnp.float32)
        return 0

    lax.fori_loop(0, 128, _zrow, 0)
    for j in range(5):
        pltpu.sync_copy(rows_a, shared.at[pl.ds(sid * 640 + j * 128, 128)])
    plsc.subcore_barrier()

    r0 = cid * ERC + sid * TROWS

    def _issue_gather(k, rows, sem):
        pltpu.async_copy(y_hbm.at[srcv.at[k]], rows, sem)

    def _wait_gather(k, rows, sem):
        pltpu.make_async_copy(y_hbm.at[srcv.at[k]], rows, sem).wait()

    def _scale_scatter(k, rows):
        def _group(gc, _):
            ew16 = ewv[k, pl.ds(gc * 16, 16)]
            for j in range(16):
                e = gc * 16 + j
                bc = lax.gather(
                    ew16, jnp.full((16, 1), j, jnp.int32),
                    lax.GatherDimensionNumbers(offset_dims=(),
                                               collapsed_slice_dims=(0,),
                                               start_index_map=(0,)),
                    slice_sizes=(1,),
                    mode=lax.GatherScatterMode.PROMISE_IN_BOUNDS)
                for c in range(8):
                    rows[e, pl.ds(c * 16, 16)] = rows[e, pl.ds(c * 16, 16)] * bc
            return 0

        lax.fori_loop(0, 8, _group, 0)
        pltpu.sync_copy(rows, shared.at[dstv.at[k]], add=True)

    def _chunk(ci, _):
        base = pl.multiple_of(r0 + ci * STG, 8)
        pltpu.sync_copy(src_hbm.at[pl.ds(base, STG)], srcv)
        pltpu.sync_copy(dst_hbm.at[pl.ds(base, STG)], dstv)
        pltpu.sync_copy(ew_hbm.at[pl.ds(base, STG)], ewv)

        _issue_gather(0, rows_a, sem_a)

        def _pair(p, _):
            k0 = p * 2
            _wait_gather(k0, rows_a, sem_a)
            _issue_gather(k0 + 1, rows_b, sem_b)
            _scale_scatter(k0, rows_a)
            _wait_gather(k0 + 1, rows_b, sem_b)

            @pl.when(k0 + 2 < STG)
            def _():
                _issue_gather(k0 + 2, rows_a, sem_a)

            _scale_scatter(k0 + 1, rows_b)
            return 0

        lax.fori_loop(0, STG // 2, _pair, 0)
        return 0

    lax.fori_loop(0, TROWS // STG, _chunk, 0)

    plsc.subcore_barrier()
    pltpu.sync_copy(shared.at[pl.ds(sid * 640, 640)],
                    out_hbm.at[cid, pl.ds(sid * 640, 640)])


_BR = 1024


def _y_mm_body(x_ref, w_ref, degp_ref, y_ref):
    deg = degp_ref[0] + degp_ref[1] + 1.0
    dv = lax.rsqrt(deg)
    y_ref[...] = jnp.dot(x_ref[...], w_ref[...],
                         preferred_element_type=jnp.float32) * dv


def _y_mm(x, w, degs):
    grid = (NP // _BR,)
    return pl.pallas_call(
        _y_mm_body,
        grid=grid,
        in_specs=[
            pl.BlockSpec((_BR, D), lambda i: (i, 0)),
            pl.BlockSpec((D, D), lambda i: (0, 0)),
            pl.BlockSpec((NC, _BR, 1), lambda i: (0, i, 0)),
        ],
        out_specs=pl.BlockSpec((_BR, D), lambda i: (i, 0)),
        out_shape=jax.ShapeDtypeStruct((N, D), jnp.float32),
    )(x, w, degs)


def _fin1_body(p0, p1, y, degs, b, g, bt, w2, hbn_ref, y2_ref):
    dv = lax.rsqrt(degs[0] + degs[1] + 1.0)
    h = jnp.maximum(dv * (p0[...] + p1[...] + y[...]) + b[...], 0.0)
    m = jnp.mean(h, axis=0, keepdims=True)
    v = jnp.mean((h - m) ** 2, axis=0, keepdims=True)
    hbn = (h - m) * lax.rsqrt(v + EPS) * g[...] + bt[...]
    hbn_ref[...] = hbn
    y2_ref[...] = jnp.dot(hbn, w2[...], preferred_element_type=jnp.float32) * dv


def _fin1(p0, p1, y, degs, b, g, bt, w2):
    return pl.pallas_call(
        _fin1_body,
        out_shape=(jax.ShapeDtypeStruct((N, D), jnp.float32),
                   jax.ShapeDtypeStruct((N, D), jnp.float32)),
    )(p0, p1, y, degs, b, g, bt, w2)


def _fin2_body(x, h1, p0, p1, y2, degs, b2, g2, bt2, fw1, fb1, fw2, fb2,
               out_ref):
    dv = lax.rsqrt(degs[0] + degs[1] + 1.0)
    h = jnp.maximum(dv * (p0[...] + p1[...] + y2[...]) + b2[...], 0.0)
    m = jnp.mean(h, axis=0, keepdims=True)
    v = jnp.mean((h - m) ** 2, axis=0, keepdims=True)
    h2bn = (h - m) * lax.rsqrt(v + EPS) * g2[...] + bt2[...]
    w = fw1[...]
    t = (jnp.dot(x[...], w[0:D], preferred_element_type=jnp.float32)
         + jnp.dot(h1[...], w[D:2 * D], preferred_element_type=jnp.float32)
         + jnp.dot(h2bn, w[2 * D:3 * D], preferred_element_type=jnp.float32)
         + fb1[...])
    t = jnp.maximum(t, 0.0)
    out = jnp.dot(t, fw2[...], preferred_element_type=jnp.float32) + fb2[...]
    out_ref[...] = jnp.maximum(out, 0.0)


def _fin2(x, h1, p0, p1, y2, degs, b2, g2, bt2, fw1, fb1, fw2, fb2):
    return pl.pallas_call(
        _fin2_body,
        out_shape=jax.ShapeDtypeStruct((N, D), jnp.float32),
    )(x, h1, p0, p1, y2, degs, b2, g2, bt2, fw1, fb1, fw2, fb2)


def kernel(x, edge_index, weight, W1, b1, W2, b2, g1, bt1, g2, bt2,
           fW1, fb1, fW2, fb2):
    npad = EP - E
    pad_idx = (jnp.arange(npad, dtype=jnp.int32) * 13) % N
    src2d = jnp.concatenate([edge_index[0], pad_idx]).reshape(ERP, 128)
    dst2d = jnp.concatenate([edge_index[1], pad_idx]).reshape(ERP, 128)
    ew2d = jnp.concatenate(
        [weight, jnp.zeros((npad,), jnp.float32)]).reshape(ERP, 128)

    degp = _deg_kernel(dst2d, ew2d)
    degs = degp.reshape(NC, NP, 1)
    degs_n = degs[:, :N]

    y1 = _y_mm(x, W1, degs)
    part1 = _msg_kernel(y1, src2d, dst2d, ew2d)
    h1, y2 = _fin1(part1[0, :N], part1[1, :N], y1, degs_n, b1, g1, bt1, W2)
    part2 = _msg_kernel(y2, src2d, dst2d, ew2d)
    out = _fin2(x, h1, part2[0, :N], part2[1, :N], y2, degs_n,
                b2, g2, bt2, fW1, fb1, fW2, fb2)
    return out
